# baseline (speedup 1.0000x reference)
import functools

import jax
import jax.numpy as jnp
from jax import lax
from jax.experimental import pallas as pl
from jax.experimental.pallas import tpu as pltpu
from jax.experimental.pallas import tpu_sc as plsc

VOCAB = 100000
EMB = 300
NOUT = 2
BATCH = 4096
HIST = 50
LANES = 16
NC, NS = 2, 16
NW = NC * NS
NSHARD = NW // NOUT
B_PER_W = BATCH // NSHARD
NGRP = B_PER_W // LANES
K_BLK = 64
K_STEPS = -(-EMB // K_BLK)
K_PAD = K_BLK * K_STEPS
VH2 = 25088
V1 = 3 * VH2
SENT = 16


def _matmul_body(wt_ref, w_ref, o0_ref, o1_ref):
    part = lax.dot_general(wt_ref[...], w_ref[...],
                           (((0,), (0,)), ((), ())),
                           preferred_element_type=jnp.float32)

    @pl.when(pl.program_id(0) == 0)
    def _():
        o0_ref[...] = part[0]
        o1_ref[...] = part[1]

    @pl.when(pl.program_id(0) > 0)
    def _():
        o0_ref[...] = o0_ref[...] + part[0]
        o1_ref[...] = o1_ref[...] + part[1]


def _project_chunk(wT, wtp, width, blk_idx):
    return pl.pallas_call(
        _matmul_body,
        grid=(K_STEPS,),
        in_specs=[
            pl.BlockSpec((K_BLK, 8), lambda i: (i, 0)),
            pl.BlockSpec((K_BLK, width), lambda i, _h=blk_idx: (i, _h)),
        ],
        out_specs=[pl.BlockSpec((width,), lambda i: (0,)),
                   pl.BlockSpec((width,), lambda i: (0,))],
        out_shape=[jax.ShapeDtypeStruct((width,), jnp.float32),
                   jax.ShapeDtypeStruct((width,), jnp.float32)],
        compiler_params=pltpu.CompilerParams(vmem_limit_bytes=56 * 2**20),
    )(wtp, wT)


def _make_pool_body(is_tail):
    def body(p0_hbm, p1_hbm, xt_hbm, *rest):
        if is_tail:
            (pin_hbm, out_hbm, tbl_v, xt_v, out_v, pin_v, scale_v,
             tbl_sem, xt_sem) = rest
        else:
            (out_hbm, tbl_v, xt_v, out_v, scale_v, tbl_sem, xt_sem) = rest
        wid = lax.axis_index("s") * NC + lax.axis_index("c")
        col = wid % NOUT
        r0 = (wid // NOUT) * B_PER_W

        xt_copy = pltpu.async_copy(xt_hbm.at[:, pl.ds(r0, B_PER_W)], xt_v,
                                   xt_sem)
        off = SENT if is_tail else 0
        width = VH2 if is_tail else V1
        if is_tail:
            tbl_v[pl.ds(0, SENT)] = jnp.zeros((SENT,), jnp.float32)
        else:
            tbl_v[pl.ds(V1, SENT)] = jnp.zeros((SENT,), jnp.float32)

        @pl.when(col == 0)
        def _():
            pltpu.async_copy(p0_hbm, tbl_v.at[pl.ds(off, width)], tbl_sem)

        @pl.when(col == 1)
        def _():
            pltpu.async_copy(p1_hbm, tbl_v.at[pl.ds(off, width)], tbl_sem)
        if is_tail:
            pltpu.sync_copy(pin_hbm.at[col, pl.ds(r0, B_PER_W)], pin_v)
        scale_v[...] = jnp.full((LANES,), 1.0 / HIST, jnp.float32)
        xt_copy.wait()
        pltpu.make_async_copy(p0_hbm, tbl_v.at[pl.ds(off, width)],
                              tbl_sem).wait()

        sent_v = jnp.full((LANES,), V1, jnp.int32)
        shift_v = jnp.full((LANES,), V1 - SENT, jnp.int32)
        zero_v = jnp.zeros((LANES,), jnp.int32)

        @pl.loop(0, NGRP)
        def _grp(g):
            idx0 = xt_v[0, pl.ds(g * LANES, LANES)]
            idx0 = (jnp.maximum(idx0 - shift_v, zero_v) if is_tail
                    else jnp.minimum(idx0, sent_v))
            acc = plsc.load_gather(tbl_v, [idx0])
            for l in range(1, HIST):
                idx = xt_v[l, pl.ds(g * LANES, LANES)]
                idx = (jnp.maximum(idx - shift_v, zero_v) if is_tail
                       else jnp.minimum(idx, sent_v))
                acc = acc + plsc.load_gather(tbl_v, [idx])
            res = acc * scale_v[...]
            if is_tail:
                res = res + pin_v[pl.ds(g * LANES, LANES)]
            out_v[pl.ds(g * LANES, LANES)] = res

        pltpu.sync_copy(out_v, out_hbm.at[col, pl.ds(r0, B_PER_W)])

    return body


@functools.cache
def _pool(is_tail):
    tbl_words = SENT + VH2 if is_tail else V1 + SENT
    scratch = [
        pltpu.VMEM((tbl_words,), jnp.float32),
        pltpu.VMEM((HIST, B_PER_W), jnp.int32),
        pltpu.VMEM((B_PER_W,), jnp.float32),
    ]
    if is_tail:
        scratch.append(pltpu.VMEM((B_PER_W,), jnp.float32))
    scratch += [
        pltpu.VMEM((LANES,), jnp.float32),
        pltpu.SemaphoreType.DMA,
        pltpu.SemaphoreType.DMA,
    ]
    return pl.kernel(
        _make_pool_body(is_tail),
        out_type=jax.ShapeDtypeStruct((NOUT, BATCH), jnp.float32),
        mesh=plsc.VectorSubcoreMesh(core_axis_name="c", subcore_axis_name="s",
                                    num_cores=NC, num_subcores=NS),
        compiler_params=pltpu.CompilerParams(use_tc_tiling_on_sc=False,
                                             needs_layout_passes=False),
        scratch_types=scratch,
    )


def kernel(x, weight, W_out):
    wtp = jnp.zeros((K_PAD, 8), jnp.float32).at[:EMB, :NOUT].set(W_out.T)
    wT = weight.T
    xt = x.astype(jnp.int32).T
    p0a, p1a = _project_chunk(wT, wtp, V1, 0)
    p0b, p1b = _project_chunk(wT, wtp, VH2, 3)
    partial = _pool(False)(p0a, p1a, xt)
    pooled = _pool(True)(p0b, p1b, xt, partial)
    return pooled.T


# speedup vs baseline: 1.0551x; 1.0551x over previous
import functools

import jax
import jax.numpy as jnp
from jax import lax
from jax.experimental import pallas as pl
from jax.experimental.pallas import tpu as pltpu
from jax.experimental.pallas import tpu_sc as plsc

VOCAB = 100000
EMB = 300
NOUT = 2
BATCH = 4096
HIST = 50
LANES = 16
NC, NS = 2, 16
NW = NC * NS
NSHARD = NW // NOUT
B_PER_W = BATCH // NSHARD
NGRP = B_PER_W // LANES
K_BLK = 64
K_STEPS = -(-EMB // K_BLK)
K_PAD = K_BLK * K_STEPS


def _matmul_body(wt_ref, w_ref, o0_ref, o1_ref):
    part = lax.dot_general(wt_ref[...], w_ref[...],
                           (((0,), (0,)), ((), ())),
                           preferred_element_type=jnp.float32)

    @pl.when(pl.program_id(0) == 0)
    def _():
        o0_ref[...] = part[0]
        o1_ref[...] = part[1]

    @pl.when(pl.program_id(0) > 0)
    def _():
        o0_ref[...] = o0_ref[...] + part[0]
        o1_ref[...] = o1_ref[...] + part[1]


def _project(wT, wtp):
    return pl.pallas_call(
        _matmul_body,
        grid=(K_STEPS,),
        in_specs=[
            pl.BlockSpec((K_BLK, 8), lambda i: (i, 0)),
            pl.BlockSpec((K_BLK, VOCAB), lambda i: (i, 0)),
        ],
        out_specs=[pl.BlockSpec((VOCAB,), lambda i: (0,)),
                   pl.BlockSpec((VOCAB,), lambda i: (0,))],
        out_shape=[jax.ShapeDtypeStruct((VOCAB,), jnp.float32),
                   jax.ShapeDtypeStruct((VOCAB,), jnp.float32)],
        compiler_params=pltpu.CompilerParams(vmem_limit_bytes=56 * 2**20),
    )(wtp, wT)


def _pool_body(p0_hbm, p1_hbm, xt_hbm, out_hbm, tbl_v, xt_v, out_v, scale_v,
               tbl_sem, tbl2_sem, xt_sem):
    wid = lax.axis_index("s") * NC + lax.axis_index("c")
    col = wid % NOUT
    r0 = (wid // NOUT) * B_PER_W

    xt_copy = pltpu.async_copy(xt_hbm.at[:, pl.ds(r0, B_PER_W)], xt_v, xt_sem)

    half = VOCAB // 2

    @pl.when(col == 0)
    def _():
        pltpu.async_copy(p0_hbm.at[pl.ds(0, half)], tbl_v.at[pl.ds(0, half)],
                         tbl_sem)
        pltpu.async_copy(p0_hbm.at[pl.ds(half, half)],
                         tbl_v.at[pl.ds(half, half)], tbl2_sem)

    @pl.when(col == 1)
    def _():
        pltpu.async_copy(p1_hbm.at[pl.ds(0, half)], tbl_v.at[pl.ds(0, half)],
                         tbl_sem)
        pltpu.async_copy(p1_hbm.at[pl.ds(half, half)],
                         tbl_v.at[pl.ds(half, half)], tbl2_sem)
    scale_v[...] = jnp.full((LANES,), 1.0 / HIST, jnp.float32)
    xt_copy.wait()
    pltpu.make_async_copy(p0_hbm.at[pl.ds(0, half)], tbl_v.at[pl.ds(0, half)],
                          tbl_sem).wait()
    pltpu.make_async_copy(p0_hbm.at[pl.ds(half, half)],
                          tbl_v.at[pl.ds(half, half)], tbl2_sem).wait()

    @pl.loop(0, NGRP)
    def _grp(g):
        idx0 = xt_v[0, pl.ds(g * LANES, LANES)]
        acc = plsc.load_gather(tbl_v, [idx0])
        for l in range(1, HIST):
            idx = xt_v[l, pl.ds(g * LANES, LANES)]
            acc = acc + plsc.load_gather(tbl_v, [idx])
        out_v[pl.ds(g * LANES, LANES)] = acc * scale_v[...]

    pltpu.sync_copy(out_v, out_hbm.at[col, pl.ds(r0, B_PER_W)])


@functools.cache
def _pool():
    return pl.kernel(
        _pool_body,
        out_type=jax.ShapeDtypeStruct((NOUT, BATCH), jnp.float32),
        mesh=plsc.VectorSubcoreMesh(core_axis_name="c", subcore_axis_name="s",
                                    num_cores=NC, num_subcores=NS),
        compiler_params=pltpu.CompilerParams(use_tc_tiling_on_sc=False,
                                             needs_layout_passes=False),
        scratch_types=[
            pltpu.VMEM((VOCAB,), jnp.float32),
            pltpu.VMEM((HIST, B_PER_W), jnp.int32),
            pltpu.VMEM((B_PER_W,), jnp.float32),
            pltpu.VMEM((LANES,), jnp.float32),
            pltpu.SemaphoreType.DMA,
            pltpu.SemaphoreType.DMA,
            pltpu.SemaphoreType.DMA,
        ],
    )


def kernel(x, weight, W_out):
    wtp = jnp.zeros((K_PAD, 8), jnp.float32).at[:EMB, :NOUT].set(W_out.T)
    p0, p1 = _project(weight.T, wtp)
    xt = x.astype(jnp.int32).T
    pooled = _pool()(p0, p1, xt)
    return pooled.T
